# baseline (speedup 1.0000x reference)
import jax
import jax.numpy as jnp
from jax import lax
from jax.experimental import pallas as pl
from jax.experimental.pallas import tpu as pltpu
from jax.experimental.pallas import tpu_sc as plsc

N_NODES = 10000
N_EDGES = 320000
D = 128

NC = 2
NS = 16
CHUNK = 96
EDGES_PER_TILE = N_EDGES // (NC * NS)
N_CHUNKS = 105
EDGES_PAD = N_CHUNKS * CHUNK
TABLE_ROWS = 10112
ROWS_PER_TILE = TABLE_ROWS // NS
ZFULL = ROWS_PER_TILE // CHUNK
ZTAIL = ROWS_PER_TILE - ZFULL * CHUNK
NB = 3


def _seg_sum_kernel(temp_hbm, src_hbm, dst_hbm, zeros_hbm, out_hbm,
                    rows0, rows1, rows2, sb0, sb1, sb2, db0, db1, db2,
                    table_sh, g0, g1, g2, s0, s1, s2, x0, x1, x2):
    c = lax.axis_index("c")
    s = lax.axis_index("s")
    wid = c * NS + s

    rows = (rows0, rows1, rows2)
    sidx = (sb0, sb1, sb2)
    dstb = (db0, db1, db2)
    gsem = (g0, g1, g2)
    ssem = (s0, s1, s2)
    xsem = (x0, x1, x2)

    pltpu.sync_copy(zeros_hbm, rows0)
    for k in range(ZFULL):
        pltpu.sync_copy(rows0,
                        table_sh.at[pl.ds(s * ROWS_PER_TILE + k * CHUNK,
                                          CHUNK)])
    pltpu.sync_copy(rows0.at[pl.ds(0, ZTAIL)],
                    table_sh.at[pl.ds(s * ROWS_PER_TILE + ZFULL * CHUNK,
                                      ZTAIL)])
    plsc.subcore_barrier()

    def load_src(i, b):
        pltpu.async_copy(src_hbm.at[wid, i], sidx[b], xsem[b])

    def wait_src(b):
        pltpu.make_async_copy(src_hbm.at[wid, 0], sidx[b], xsem[b]).wait()

    def load_dst(i, b):
        pltpu.async_copy(dst_hbm.at[wid, i], dstb[b], gsem[b])

    def issue_gather(b):
        pltpu.async_copy(temp_hbm.at[sidx[b]], rows[b], gsem[b])

    def wait_gather(b):
        pltpu.make_async_copy(dst_hbm.at[wid, 0], dstb[b], gsem[b]).wait()
        pltpu.make_async_copy(temp_hbm.at[pl.ds(0, CHUNK)],
                              rows[b], gsem[b]).wait()

    def start_scatter(b):
        pltpu.async_copy(rows[b], table_sh.at[dstb[b]], ssem[b], add=True)

    def wait_scatter(b):
        pltpu.make_async_copy(rows[b], table_sh.at[dstb[b]], ssem[b]).wait()

    for b in range(NB):
        load_src(b, b)
    for b in range(2):
        load_dst(b, b)
        wait_src(b)
        issue_gather(b)

    def step(i, b, first, last):
        wait_gather(b)
        start_scatter(b)
        if not first:
            wait_scatter((b + 2) % NB)
        if last:
            return
        bn = (b + 2) % NB

        def load_src_ahead():
            load_src(i + NB, b)

        def issue_ahead():
            load_dst(i + 2, bn)
            wait_src(bn)
            issue_gather(bn)

        if isinstance(i, int):
            if i + NB < N_CHUNKS:
                load_src_ahead()
            if i + 2 < N_CHUNKS:
                issue_ahead()
        else:
            pl.when(i + NB < N_CHUNKS)(load_src_ahead)
            pl.when(i + 2 < N_CHUNKS)(issue_ahead)

    step(0, 0, True, False)

    def triple(k, carry):
        i = 3 * k
        step(i + 1, 1, False, False)
        step(i + 2, 2, False, False)
        step(i + 3, 0, False, False)
        return carry

    lax.fori_loop(0, (N_CHUNKS - 3) // 3, triple, 0)
    step(N_CHUNKS - 2, (N_CHUNKS - 2) % NB, False, False)
    step(N_CHUNKS - 1, (N_CHUNKS - 1) % NB, False, True)
    wait_scatter((N_CHUNKS - 1) % NB)

    plsc.subcore_barrier()

    for k in range(ZFULL):
        r0 = s * ROWS_PER_TILE + k * CHUNK
        pltpu.sync_copy(table_sh.at[pl.ds(r0, CHUNK)], rows0)
        pltpu.sync_copy(rows0, out_hbm.at[c, pl.ds(r0, CHUNK)])
    r0 = s * ROWS_PER_TILE + ZFULL * CHUNK
    pltpu.sync_copy(table_sh.at[pl.ds(r0, ZTAIL)], rows0.at[pl.ds(0, ZTAIL)])
    pltpu.sync_copy(rows0.at[pl.ds(0, ZTAIL)], out_hbm.at[c, pl.ds(r0, ZTAIL)])


def _seg_sum(temp, src3, dst3, zeros_stage):
    mesh = plsc.VectorSubcoreMesh(core_axis_name="c", subcore_axis_name="s",
                                  num_cores=NC, num_subcores=NS)
    kern = pl.kernel(
        _seg_sum_kernel,
        out_type=jax.ShapeDtypeStruct((NC, TABLE_ROWS, D), jnp.float32),
        mesh=mesh,
        scratch_types=(
            [pltpu.VMEM((CHUNK, D), jnp.float32) for _ in range(NB)]
            + [pltpu.VMEM((CHUNK,), jnp.int32) for _ in range(2 * NB)]
            + [pltpu.VMEM_SHARED((TABLE_ROWS, D), jnp.float32)]
            + [pltpu.SemaphoreType.DMA for _ in range(3 * NB)]
        ),
    )
    return kern(temp, src3, dst3, zeros_stage)


ROW_BLK = 1000


def _lin_body(t_ref, a0_ref, a1_ref, w_ref, b_ref, sc_ref):
    x = sc_ref[...] * t_ref[...] + a0_ref[0] + a1_ref[0]
    y = jnp.dot(x, w_ref[...], preferred_element_type=jnp.float32)
    return jnp.maximum(y + b_ref[...], 0.0)


def _lin_kernel(t_ref, a0_ref, a1_ref, w_ref, b_ref, sc_ref, o_ref):
    o_ref[...] = _lin_body(t_ref, a0_ref, a1_ref, w_ref, b_ref, sc_ref)


_LIN_IN_SPECS = [
    pl.BlockSpec((ROW_BLK, D), lambda i: (i, 0)),
    pl.BlockSpec((1, ROW_BLK, D), lambda i: (0, i, 0)),
    pl.BlockSpec((1, ROW_BLK, D), lambda i: (1, i, 0)),
    pl.BlockSpec((D, D), lambda i: (0, 0)),
    pl.BlockSpec((1, D), lambda i: (0, 0)),
    pl.BlockSpec((1, D), lambda i: (0, 0)),
]


def _lin_layer(temp, agg2, w, b_row, scale_row):
    return pl.pallas_call(
        _lin_kernel,
        grid=(N_NODES // ROW_BLK,),
        in_specs=_LIN_IN_SPECS,
        out_specs=pl.BlockSpec((ROW_BLK, D), lambda i: (i, 0)),
        out_shape=jax.ShapeDtypeStruct((N_NODES, D), jnp.float32),
    )(temp, agg2, agg2, w, b_row, scale_row)


def _lin2_kernel(t_ref, a0_ref, a1_ref, w_ref, b_ref, sc_ref, o_ref, acc_ref):
    i = pl.program_id(0)

    @pl.when(i == 0)
    def _():
        acc_ref[...] = jnp.zeros_like(acc_ref)

    y = _lin_body(t_ref, a0_ref, a1_ref, w_ref, b_ref, sc_ref)
    acc_ref[...] += jnp.sum(y, axis=0, keepdims=True)

    @pl.when(i == pl.num_programs(0) - 1)
    def _():
        o_ref[...] = jnp.maximum(acc_ref[...] * (1.0 / N_NODES), 0.0)


def _lin_layer2_readout(temp, agg2, w, b_row, scale_row):
    return pl.pallas_call(
        _lin2_kernel,
        grid=(N_NODES // ROW_BLK,),
        in_specs=_LIN_IN_SPECS,
        out_specs=pl.BlockSpec((1, D), lambda i: (0, 0)),
        out_shape=jax.ShapeDtypeStruct((1, D), jnp.float32),
        scratch_shapes=[pltpu.VMEM((1, D), jnp.float32)],
    )(temp, agg2, agg2, w, b_row, scale_row)


def kernel(X, h, epsilon, edge_index, W1, b1, W2, b2, eps0, eps1):
    temp = jnp.concatenate([X, epsilon, h], axis=1)
    pad = ((0, 0), (0, EDGES_PAD - EDGES_PER_TILE))
    src = jnp.pad(edge_index[0].reshape(NC * NS, EDGES_PER_TILE), pad
                  ).reshape(NC * NS, N_CHUNKS, CHUNK)
    dst = jnp.pad(edge_index[1].reshape(NC * NS, EDGES_PER_TILE), pad,
                  constant_values=N_NODES).reshape(NC * NS, N_CHUNKS, CHUNK)
    zeros_stage = jnp.zeros((CHUNK, D), jnp.float32)
    b1r = b1.reshape(1, D)
    b2r = b2.reshape(1, D)
    sc0 = jnp.full((1, D), 1.0, jnp.float32) + eps0
    sc1 = jnp.full((1, D), 1.0, jnp.float32) + eps1

    agg2 = _seg_sum(temp, src, dst, zeros_stage)
    y1 = _lin_layer(temp, agg2, W1, b1r, sc0)
    agg2 = _seg_sum(y1, src, dst, zeros_stage)
    out = _lin_layer2_readout(y1, agg2, W2, b2r, sc1)
    return (out.reshape(D), epsilon)


# speedup vs baseline: 1.0838x; 1.0838x over previous
import jax
import jax.numpy as jnp
from jax import lax
from jax.experimental import pallas as pl
from jax.experimental.pallas import tpu as pltpu
from jax.experimental.pallas import tpu_sc as plsc

N_NODES = 10000
N_EDGES = 320000
D = 128

NC = 2
NS = 16
CHUNK = 64
EDGES_PER_TILE = N_EDGES // (NC * NS)
N_CHUNKS = 157
EDGES_PAD = N_CHUNKS * CHUNK
TABLE_ROWS = 10240
ROWS_PER_TILE = TABLE_ROWS // NS
NB = 4


def _seg_sum_kernel(temp_hbm, src_hbm, dst_hbm, zeros_hbm, out_hbm,
                    rows0, rows1, rows2, rows3, sb0, sb1, sb2, sb3,
                    db0, db1, db2, db3, table_sh,
                    g0, g1, g2, g3, s0, s1, s2, s3, x0, x1, x2, x3):
    c = lax.axis_index("c")
    s = lax.axis_index("s")
    wid = c * NS + s

    rows = (rows0, rows1, rows2, rows3)
    sidx = (sb0, sb1, sb2, sb3)
    dstb = (db0, db1, db2, db3)
    gsem = (g0, g1, g2, g3)
    ssem = (s0, s1, s2, s3)
    xsem = (x0, x1, x2, x3)

    pltpu.sync_copy(zeros_hbm, rows0)
    for k in range(ROWS_PER_TILE // CHUNK):
        pltpu.sync_copy(rows0,
                        table_sh.at[pl.ds(s * ROWS_PER_TILE + k * CHUNK,
                                          CHUNK)])
    plsc.subcore_barrier()

    def _off(i):
        return pl.multiple_of(wid * EDGES_PAD + i * CHUNK, 8)

    def load_src(i, b):
        pltpu.async_copy(src_hbm.at[pl.ds(_off(i), CHUNK)], sidx[b], xsem[b])

    def wait_src(b):
        pltpu.make_async_copy(src_hbm.at[pl.ds(0, CHUNK)],
                              sidx[b], xsem[b]).wait()

    def load_dst(i, b):
        pltpu.async_copy(dst_hbm.at[pl.ds(_off(i), CHUNK)], dstb[b], gsem[b])

    def issue_gather(b):
        pltpu.async_copy(temp_hbm.at[sidx[b]], rows[b], gsem[b])

    def wait_gather(b):
        pltpu.make_async_copy(dst_hbm.at[pl.ds(0, CHUNK)],
                              dstb[b], gsem[b]).wait()
        pltpu.make_async_copy(temp_hbm.at[pl.ds(0, CHUNK)],
                              rows[b], gsem[b]).wait()

    def start_scatter(b):
        pltpu.async_copy(rows[b], table_sh.at[dstb[b]], ssem[b], add=True)

    def wait_scatter(b):
        pltpu.make_async_copy(rows[b], table_sh.at[dstb[b]], ssem[b]).wait()

    for b in range(NB):
        load_src(b, b)
    for b in range(2):
        load_dst(b, b)
        wait_src(b)
        issue_gather(b)

    def step(i, b, first, last):
        wait_gather(b)
        start_scatter(b)
        if not first:
            wait_scatter((b + 2) % NB)
        if last:
            return
        bn = (b + 2) % NB

        def load_src_ahead():
            load_src(i + NB, b)

        def issue_ahead():
            load_dst(i + 2, bn)
            wait_src(bn)
            issue_gather(bn)

        if isinstance(i, int):
            if i + NB < N_CHUNKS:
                load_src_ahead()
            if i + 2 < N_CHUNKS:
                issue_ahead()
        else:
            pl.when(i + NB < N_CHUNKS)(load_src_ahead)
            pl.when(i + 2 < N_CHUNKS)(issue_ahead)

    step(0, 0, True, False)
    step(1, 1, True, False)

    def quad(k, carry):
        i = 4 * k
        step(i + 2, 2, False, False)
        step(i + 3, 3, False, False)
        step(i + 4, 0, False, False)
        step(i + 5, 1, False, False)
        return carry

    lax.fori_loop(0, (N_CHUNKS - 5) // 4, quad, 0)
    step(N_CHUNKS - 3, (N_CHUNKS - 3) % NB, False, False)
    step(N_CHUNKS - 2, (N_CHUNKS - 2) % NB, False, False)
    step(N_CHUNKS - 1, (N_CHUNKS - 1) % NB, False, True)
    wait_scatter((N_CHUNKS - 2) % NB)
    wait_scatter((N_CHUNKS - 1) % NB)

    plsc.subcore_barrier()

    for k in range(ROWS_PER_TILE // CHUNK):
        r0 = s * ROWS_PER_TILE + k * CHUNK
        pltpu.sync_copy(table_sh.at[pl.ds(r0, CHUNK)], rows0)
        pltpu.sync_copy(rows0, out_hbm.at[c, pl.ds(r0, CHUNK)])


def _seg_sum(temp, src3, dst3, zeros_stage):
    mesh = plsc.VectorSubcoreMesh(core_axis_name="c", subcore_axis_name="s",
                                  num_cores=NC, num_subcores=NS)
    kern = pl.kernel(
        _seg_sum_kernel,
        out_type=jax.ShapeDtypeStruct((NC, TABLE_ROWS, D), jnp.float32),
        mesh=mesh,
        scratch_types=(
            [pltpu.VMEM((CHUNK, D), jnp.float32) for _ in range(NB)]
            + [pltpu.VMEM((CHUNK,), jnp.int32) for _ in range(2 * NB)]
            + [pltpu.VMEM_SHARED((TABLE_ROWS, D), jnp.float32)]
            + [pltpu.SemaphoreType.DMA for _ in range(3 * NB)]
        ),
    )
    return kern(temp, src3, dst3, zeros_stage)


ROW_BLK = 1000


def _lin_body(t_ref, a0_ref, a1_ref, w_ref, b_ref, sc_ref):
    x = sc_ref[...] * t_ref[...] + a0_ref[0] + a1_ref[0]
    y = jnp.dot(x, w_ref[...], preferred_element_type=jnp.float32)
    return jnp.maximum(y + b_ref[...], 0.0)


def _lin_kernel(t_ref, a0_ref, a1_ref, w_ref, b_ref, sc_ref, o_ref):
    o_ref[...] = _lin_body(t_ref, a0_ref, a1_ref, w_ref, b_ref, sc_ref)


_LIN_IN_SPECS = [
    pl.BlockSpec((ROW_BLK, D), lambda i: (i, 0)),
    pl.BlockSpec((1, ROW_BLK, D), lambda i: (0, i, 0)),
    pl.BlockSpec((1, ROW_BLK, D), lambda i: (1, i, 0)),
    pl.BlockSpec((D, D), lambda i: (0, 0)),
    pl.BlockSpec((1, D), lambda i: (0, 0)),
    pl.BlockSpec((1, D), lambda i: (0, 0)),
]


def _lin_layer(temp, agg2, w, b_row, scale_row):
    return pl.pallas_call(
        _lin_kernel,
        grid=(N_NODES // ROW_BLK,),
        in_specs=_LIN_IN_SPECS,
        out_specs=pl.BlockSpec((ROW_BLK, D), lambda i: (i, 0)),
        out_shape=jax.ShapeDtypeStruct((N_NODES, D), jnp.float32),
    )(temp, agg2, agg2, w, b_row, scale_row)


def _lin2_kernel(t_ref, a0_ref, a1_ref, w_ref, b_ref, sc_ref, o_ref, acc_ref):
    i = pl.program_id(0)

    @pl.when(i == 0)
    def _():
        acc_ref[...] = jnp.zeros_like(acc_ref)

    y = _lin_body(t_ref, a0_ref, a1_ref, w_ref, b_ref, sc_ref)
    acc_ref[...] += jnp.sum(y, axis=0, keepdims=True)

    @pl.when(i == pl.num_programs(0) - 1)
    def _():
        o_ref[...] = jnp.maximum(acc_ref[...] * (1.0 / N_NODES), 0.0)


def _lin_layer2_readout(temp, agg2, w, b_row, scale_row):
    return pl.pallas_call(
        _lin2_kernel,
        grid=(N_NODES // ROW_BLK,),
        in_specs=_LIN_IN_SPECS,
        out_specs=pl.BlockSpec((1, D), lambda i: (0, 0)),
        out_shape=jax.ShapeDtypeStruct((1, D), jnp.float32),
        scratch_shapes=[pltpu.VMEM((1, D), jnp.float32)],
    )(temp, agg2, agg2, w, b_row, scale_row)


def kernel(X, h, epsilon, edge_index, W1, b1, W2, b2, eps0, eps1):
    temp = jnp.concatenate([X, epsilon, h], axis=1)
    pad = ((0, 0), (0, EDGES_PAD - EDGES_PER_TILE))
    src = jnp.pad(edge_index[0].reshape(NC * NS, EDGES_PER_TILE), pad
                  ).reshape(NC * NS * EDGES_PAD)
    dst = jnp.pad(edge_index[1].reshape(NC * NS, EDGES_PER_TILE), pad,
                  constant_values=N_NODES).reshape(NC * NS * EDGES_PAD)
    zeros_stage = jnp.zeros((CHUNK, D), jnp.float32)
    b1r = b1.reshape(1, D)
    b2r = b2.reshape(1, D)
    sc0 = jnp.full((1, D), 1.0, jnp.float32) + eps0
    sc1 = jnp.full((1, D), 1.0, jnp.float32) + eps1

    agg2 = _seg_sum(temp, src, dst, zeros_stage)
    y1 = _lin_layer(temp, agg2, W1, b1r, sc0)
    agg2 = _seg_sum(y1, src, dst, zeros_stage)
    out = _lin_layer2_readout(y1, agg2, W2, b2r, sc1)
    return (out.reshape(D), epsilon)


# speedup vs baseline: 1.7800x; 1.6424x over previous
import jax
import jax.numpy as jnp
from jax import lax
from jax.experimental import pallas as pl
from jax.experimental.pallas import tpu as pltpu
from jax.experimental.pallas import tpu_sc as plsc

N_NODES = 10000
N_EDGES = 320000
D = 128

NC = 2
NS = 16
CHUNK = 80
EDGES_PER_TILE = N_EDGES // (NC * NS)
N_CHUNKS = EDGES_PER_TILE // CHUNK
TABLE_ROWS = 10240
ROWS_PER_TILE = TABLE_ROWS // NS
NB = 3


def _seg_sum_kernel(temp_hbm, edges_hbm, zeros_hbm, out_hbm,
                    rows0, rows1, rows2, sb0, sb1, sb2, db0, db1, db2,
                    table_sh, g0, g1, g2, s0, s1, s2, x0, x1, x2):
    c = lax.axis_index("c")
    s = lax.axis_index("s")
    wid = c * NS + s

    rows = (rows0, rows1, rows2)
    sidx = (sb0, sb1, sb2)
    dstb = (db0, db1, db2)
    gsem = (g0, g1, g2)
    ssem = (s0, s1, s2)
    xsem = (x0, x1, x2)

    pltpu.sync_copy(zeros_hbm, rows0)
    for k in range(ROWS_PER_TILE // CHUNK):
        pltpu.sync_copy(rows0,
                        table_sh.at[pl.ds(s * ROWS_PER_TILE + k * CHUNK,
                                          CHUNK)])
    plsc.subcore_barrier()

    def _off(i, base):
        return pl.multiple_of(base + wid * EDGES_PER_TILE + i * CHUNK, 8)

    def load_src(i, b):
        pltpu.async_copy(edges_hbm.at[pl.ds(_off(i, 0), CHUNK)],
                         sidx[b], xsem[b])

    def wait_src(b):
        pltpu.make_async_copy(edges_hbm.at[pl.ds(0, CHUNK)],
                              sidx[b], xsem[b]).wait()

    def load_dst(i, b):
        pltpu.async_copy(edges_hbm.at[pl.ds(_off(i, N_EDGES), CHUNK)],
                         dstb[b], gsem[b])

    def issue_gather(b):
        pltpu.async_copy(temp_hbm.at[sidx[b]], rows[b], gsem[b])

    def wait_gather(b):
        pltpu.make_async_copy(edges_hbm.at[pl.ds(0, CHUNK)],
                              dstb[b], gsem[b]).wait()
        pltpu.make_async_copy(temp_hbm.at[pl.ds(0, CHUNK)],
                              rows[b], gsem[b]).wait()

    def start_scatter(b):
        pltpu.async_copy(rows[b], table_sh.at[dstb[b]], ssem[b], add=True)

    def wait_scatter(b):
        pltpu.make_async_copy(rows[b], table_sh.at[dstb[b]], ssem[b]).wait()

    for b in range(NB):
        load_src(b, b)
    for b in range(2):
        load_dst(b, b)
        wait_src(b)
        issue_gather(b)

    def step(i, b, first, last):
        wait_gather(b)
        start_scatter(b)
        if not first:
            wait_scatter((b + 2) % NB)
        if last:
            return
        bn = (b + 2) % NB

        def load_src_ahead():
            load_src(i + NB, b)

        def issue_ahead():
            load_dst(i + 2, bn)
            wait_src(bn)
            issue_gather(bn)

        if isinstance(i, int):
            if i + NB < N_CHUNKS:
                load_src_ahead()
            if i + 2 < N_CHUNKS:
                issue_ahead()
        else:
            pl.when(i + NB < N_CHUNKS)(load_src_ahead)
            pl.when(i + 2 < N_CHUNKS)(issue_ahead)

    step(0, 0, True, False)

    def triple(k, carry):
        i = 3 * k
        step(i + 1, 1, False, False)
        step(i + 2, 2, False, False)
        step(i + 3, 0, False, False)
        return carry

    lax.fori_loop(0, (N_CHUNKS - 2) // 3, triple, 0)
    step(N_CHUNKS - 1, (N_CHUNKS - 1) % NB, False, True)
    wait_scatter((N_CHUNKS - 1) % NB)

    plsc.subcore_barrier()

    for k in range(ROWS_PER_TILE // CHUNK):
        r0 = s * ROWS_PER_TILE + k * CHUNK
        pltpu.sync_copy(table_sh.at[pl.ds(r0, CHUNK)], rows0)
        pltpu.sync_copy(rows0, out_hbm.at[c, pl.ds(r0, CHUNK)])


def _seg_sum(temp, edges_flat, zeros_stage):
    mesh = plsc.VectorSubcoreMesh(core_axis_name="c", subcore_axis_name="s",
                                  num_cores=NC, num_subcores=NS)
    kern = pl.kernel(
        _seg_sum_kernel,
        out_type=jax.ShapeDtypeStruct((NC, TABLE_ROWS, D), jnp.float32),
        mesh=mesh,
        scratch_types=(
            [pltpu.VMEM((CHUNK, D), jnp.float32) for _ in range(NB)]
            + [pltpu.VMEM((CHUNK,), jnp.int32) for _ in range(2 * NB)]
            + [pltpu.VMEM_SHARED((TABLE_ROWS, D), jnp.float32)]
            + [pltpu.SemaphoreType.DMA for _ in range(3 * NB)]
        ),
    )
    return kern(temp, edges_flat, zeros_stage)


ROW_BLK = 2000
D_X = 96
D_NZ = 16


def _prep_kernel(x_ref, e_ref, h_ref, o_ref):
    o_ref[:, :D_X] = x_ref[...]
    o_ref[:, D_X:D_X + D_NZ] = e_ref[...]
    o_ref[:, D_X + D_NZ:] = h_ref[...]


def _prep_temp(X, epsilon, h):
    return pl.pallas_call(
        _prep_kernel,
        grid=(N_NODES // ROW_BLK,),
        in_specs=[
            pl.BlockSpec((ROW_BLK, D_X), lambda i: (i, 0)),
            pl.BlockSpec((ROW_BLK, D_NZ), lambda i: (i, 0)),
            pl.BlockSpec((ROW_BLK, D_NZ), lambda i: (i, 0)),
        ],
        out_specs=pl.BlockSpec((ROW_BLK, D), lambda i: (i, 0)),
        out_shape=jax.ShapeDtypeStruct((N_NODES, D), jnp.float32),
    )(X, epsilon, h)


def _lin_body(t_ref, a0_ref, a1_ref, w_ref, b_ref, sc_ref):
    x = sc_ref[...] * t_ref[...] + a0_ref[0] + a1_ref[0]
    y = jnp.dot(x, w_ref[...], preferred_element_type=jnp.float32)
    return jnp.maximum(y + b_ref[...], 0.0)


def _lin_kernel(t_ref, a0_ref, a1_ref, w_ref, b_ref, sc_ref, o_ref):
    o_ref[...] = _lin_body(t_ref, a0_ref, a1_ref, w_ref, b_ref, sc_ref)


_LIN_IN_SPECS = [
    pl.BlockSpec((ROW_BLK, D), lambda i: (i, 0)),
    pl.BlockSpec((1, ROW_BLK, D), lambda i: (0, i, 0)),
    pl.BlockSpec((1, ROW_BLK, D), lambda i: (1, i, 0)),
    pl.BlockSpec((D, D), lambda i: (0, 0)),
    pl.BlockSpec((1, D), lambda i: (0, 0)),
    pl.BlockSpec((1, D), lambda i: (0, 0)),
]


def _lin_layer(temp, agg2, w, b_row, scale_row):
    return pl.pallas_call(
        _lin_kernel,
        grid=(N_NODES // ROW_BLK,),
        in_specs=_LIN_IN_SPECS,
        out_specs=pl.BlockSpec((ROW_BLK, D), lambda i: (i, 0)),
        out_shape=jax.ShapeDtypeStruct((N_NODES, D), jnp.float32),
    )(temp, agg2, agg2, w, b_row, scale_row)


def _lin2_kernel(t_ref, a0_ref, a1_ref, w_ref, b_ref, sc_ref, o_ref, acc_ref):
    i = pl.program_id(0)

    @pl.when(i == 0)
    def _():
        acc_ref[...] = jnp.zeros_like(acc_ref)

    y = _lin_body(t_ref, a0_ref, a1_ref, w_ref, b_ref, sc_ref)
    acc_ref[...] += jnp.sum(y, axis=0, keepdims=True)

    @pl.when(i == pl.num_programs(0) - 1)
    def _():
        o_ref[...] = jnp.maximum(acc_ref[...] * (1.0 / N_NODES), 0.0)


def _lin_layer2_readout(temp, agg2, w, b_row, scale_row):
    return pl.pallas_call(
        _lin2_kernel,
        grid=(N_NODES // ROW_BLK,),
        in_specs=_LIN_IN_SPECS,
        out_specs=pl.BlockSpec((1, D), lambda i: (0, 0)),
        out_shape=jax.ShapeDtypeStruct((1, D), jnp.float32),
        scratch_shapes=[pltpu.VMEM((1, D), jnp.float32)],
    )(temp, agg2, agg2, w, b_row, scale_row)


def kernel(X, h, epsilon, edge_index, W1, b1, W2, b2, eps0, eps1):
    temp = _prep_temp(X, epsilon, h)
    edges_flat = edge_index.reshape(2 * N_EDGES)
    zeros_stage = jnp.zeros((CHUNK, D), jnp.float32)
    b1r = b1.reshape(1, D)
    b2r = b2.reshape(1, D)
    sc0 = jnp.full((1, D), 1.0, jnp.float32) + eps0
    sc1 = jnp.full((1, D), 1.0, jnp.float32) + eps1

    agg2 = _seg_sum(temp, edges_flat, zeros_stage)
    y1 = _lin_layer(temp, agg2, W1, b1r, sc0)
    agg2 = _seg_sum(y1, edges_flat, zeros_stage)
    out = _lin_layer2_readout(y1, agg2, W2, b2r, sc1)
    return (out.reshape(D), epsilon)


# speedup vs baseline: 1.8511x; 1.0399x over previous
import jax
import jax.numpy as jnp
from jax import lax
from jax.experimental import pallas as pl
from jax.experimental.pallas import tpu as pltpu
from jax.experimental.pallas import tpu_sc as plsc

N_NODES = 10000
N_EDGES = 320000
D = 128

NC = 2
NS = 16
CHUNK = 80
EDGES_PER_TILE = N_EDGES // (NC * NS)
N_CHUNKS = EDGES_PER_TILE // CHUNK
TABLE_ROWS = 10240
ROWS_PER_TILE = TABLE_ROWS // NS
NB = 3


def _seg_sum_kernel(temp_hbm, edges_hbm, zeros_hbm, out_hbm,
                    rows0, rows1, rows2, sb0, sb1, sb2, db0, db1, db2,
                    table_sh, g0, g1, g2, s0, s1, s2, x0, x1, x2):
    c = lax.axis_index("c")
    s = lax.axis_index("s")
    wid = c * NS + s

    rows = (rows0, rows1, rows2)
    sidx = (sb0, sb1, sb2)
    dstb = (db0, db1, db2)
    gsem = (g0, g1, g2)
    ssem = (s0, s1, s2)
    xsem = (x0, x1, x2)

    pltpu.sync_copy(zeros_hbm, rows0)
    for k in range(ROWS_PER_TILE // CHUNK):
        pltpu.sync_copy(rows0,
                        table_sh.at[pl.ds(s * ROWS_PER_TILE + k * CHUNK,
                                          CHUNK)])
    plsc.subcore_barrier()

    def _off(i, base):
        return pl.multiple_of(base + wid * EDGES_PER_TILE + i * CHUNK, 8)

    def load_src(i, b):
        pltpu.async_copy(edges_hbm.at[pl.ds(_off(i, 0), CHUNK)],
                         sidx[b], xsem[b])

    def wait_src(b):
        pltpu.make_async_copy(edges_hbm.at[pl.ds(0, CHUNK)],
                              sidx[b], xsem[b]).wait()

    def load_dst(i, b):
        pltpu.async_copy(edges_hbm.at[pl.ds(_off(i, N_EDGES), CHUNK)],
                         dstb[b], gsem[b])

    def issue_gather(b):
        pltpu.async_copy(temp_hbm.at[sidx[b]], rows[b], gsem[b])

    def wait_gather(b):
        pltpu.make_async_copy(edges_hbm.at[pl.ds(0, CHUNK)],
                              dstb[b], gsem[b]).wait()
        pltpu.make_async_copy(temp_hbm.at[pl.ds(0, CHUNK)],
                              rows[b], gsem[b]).wait()

    def start_scatter(b):
        pltpu.async_copy(rows[b], table_sh.at[dstb[b]], ssem[b], add=True)

    def wait_scatter(b):
        pltpu.make_async_copy(rows[b], table_sh.at[dstb[b]], ssem[b]).wait()

    for b in range(NB):
        load_src(b, b)
    for b in range(2):
        load_dst(b, b)
        wait_src(b)
        issue_gather(b)

    def step(i, b, first, last):
        wait_gather(b)
        start_scatter(b)
        if not first:
            wait_scatter((b + 2) % NB)
        if last:
            return
        bn = (b + 2) % NB

        def load_src_ahead():
            load_src(i + NB, b)

        def issue_ahead():
            load_dst(i + 2, bn)
            wait_src(bn)
            issue_gather(bn)

        if isinstance(i, int):
            if i + NB < N_CHUNKS:
                load_src_ahead()
            if i + 2 < N_CHUNKS:
                issue_ahead()
        else:
            pl.when(i + NB < N_CHUNKS)(load_src_ahead)
            pl.when(i + 2 < N_CHUNKS)(issue_ahead)

    step(0, 0, True, False)

    def triple(k, carry):
        i = 3 * k
        step(i + 1, 1, False, False)
        step(i + 2, 2, False, False)
        step(i + 3, 0, False, False)
        return carry

    lax.fori_loop(0, (N_CHUNKS - 2) // 3, triple, 0)
    step(N_CHUNKS - 1, (N_CHUNKS - 1) % NB, False, True)
    wait_scatter((N_CHUNKS - 1) % NB)

    plsc.subcore_barrier()

    for k in range(ROWS_PER_TILE // CHUNK):
        r0 = s * ROWS_PER_TILE + k * CHUNK
        pltpu.sync_copy(table_sh.at[pl.ds(r0, CHUNK)], rows0)
        pltpu.sync_copy(rows0, out_hbm.at[c, pl.ds(r0, CHUNK)])


def _seg_sum(temp, edges_flat, zeros_stage):
    mesh = plsc.VectorSubcoreMesh(core_axis_name="c", subcore_axis_name="s",
                                  num_cores=NC, num_subcores=NS)
    kern = pl.kernel(
        _seg_sum_kernel,
        out_type=jax.ShapeDtypeStruct((NC, TABLE_ROWS, D), jnp.float32),
        mesh=mesh,
        scratch_types=(
            [pltpu.VMEM((CHUNK, D), jnp.float32) for _ in range(NB)]
            + [pltpu.VMEM((CHUNK,), jnp.int32) for _ in range(2 * NB)]
            + [pltpu.VMEM_SHARED((TABLE_ROWS, D), jnp.float32)]
            + [pltpu.SemaphoreType.DMA for _ in range(3 * NB)]
        ),
    )
    return kern(temp, edges_flat, zeros_stage)


ROW_BLK = 2000
D_X = 96
D_NZ = 16


def _prep_kernel(xt_ref, et_ref, ht_ref, o_ref):
    x = jnp.swapaxes(xt_ref[...], 0, 1)
    e = jnp.swapaxes(et_ref[...], 0, 1)
    h = jnp.swapaxes(ht_ref[...], 0, 1)
    o_ref[...] = jnp.concatenate([x, e, h], axis=1)


def _prep_temp(X, epsilon, h):
    return pl.pallas_call(
        _prep_kernel,
        out_shape=jax.ShapeDtypeStruct((N_NODES, D), jnp.float32),
    )(X.T, epsilon.T, h.T)


def _lin_body(t_ref, a0_ref, a1_ref, w_ref, b_ref, sc_ref):
    x = sc_ref[...] * t_ref[...] + a0_ref[0] + a1_ref[0]
    y = jnp.dot(x, w_ref[...], preferred_element_type=jnp.float32)
    return jnp.maximum(y + b_ref[...], 0.0)


def _lin_kernel(t_ref, a0_ref, a1_ref, w_ref, b_ref, sc_ref, o_ref):
    o_ref[...] = _lin_body(t_ref, a0_ref, a1_ref, w_ref, b_ref, sc_ref)


_LIN_IN_SPECS = [
    pl.BlockSpec((ROW_BLK, D), lambda i: (i, 0)),
    pl.BlockSpec((1, ROW_BLK, D), lambda i: (0, i, 0)),
    pl.BlockSpec((1, ROW_BLK, D), lambda i: (1, i, 0)),
    pl.BlockSpec((D, D), lambda i: (0, 0)),
    pl.BlockSpec((1, D), lambda i: (0, 0)),
    pl.BlockSpec((1, D), lambda i: (0, 0)),
]


def _lin_layer(temp, agg2, w, b_row, scale_row):
    return pl.pallas_call(
        _lin_kernel,
        grid=(N_NODES // ROW_BLK,),
        in_specs=_LIN_IN_SPECS,
        out_specs=pl.BlockSpec((ROW_BLK, D), lambda i: (i, 0)),
        out_shape=jax.ShapeDtypeStruct((N_NODES, D), jnp.float32),
    )(temp, agg2, agg2, w, b_row, scale_row)


def _lin2_kernel(t_ref, a0_ref, a1_ref, w_ref, b_ref, sc_ref, o_ref, acc_ref):
    i = pl.program_id(0)

    @pl.when(i == 0)
    def _():
        acc_ref[...] = jnp.zeros_like(acc_ref)

    y = _lin_body(t_ref, a0_ref, a1_ref, w_ref, b_ref, sc_ref)
    acc_ref[...] += jnp.sum(y, axis=0, keepdims=True)

    @pl.when(i == pl.num_programs(0) - 1)
    def _():
        o_ref[...] = jnp.maximum(acc_ref[...] * (1.0 / N_NODES), 0.0)


def _lin_layer2_readout(temp, agg2, w, b_row, scale_row):
    return pl.pallas_call(
        _lin2_kernel,
        grid=(N_NODES // ROW_BLK,),
        in_specs=_LIN_IN_SPECS,
        out_specs=pl.BlockSpec((1, D), lambda i: (0, 0)),
        out_shape=jax.ShapeDtypeStruct((1, D), jnp.float32),
        scratch_shapes=[pltpu.VMEM((1, D), jnp.float32)],
    )(temp, agg2, agg2, w, b_row, scale_row)


def kernel(X, h, epsilon, edge_index, W1, b1, W2, b2, eps0, eps1):
    temp = _prep_temp(X, epsilon, h)
    edges_flat = edge_index.reshape(2 * N_EDGES)
    zeros_stage = jnp.zeros((CHUNK, D), jnp.float32)
    b1r = b1.reshape(1, D)
    b2r = b2.reshape(1, D)
    sc0 = jnp.full((1, D), 1.0, jnp.float32) + eps0
    sc1 = jnp.full((1, D), 1.0, jnp.float32) + eps1

    agg2 = _seg_sum(temp, edges_flat, zeros_stage)
    y1 = _lin_layer(temp, agg2, W1, b1r, sc0)
    agg2 = _seg_sum(y1, edges_flat, zeros_stage)
    out = _lin_layer2_readout(y1, agg2, W2, b2r, sc1)
    return (out.reshape(D), epsilon)


# speedup vs baseline: 1.8849x; 1.0183x over previous
import jax
import jax.numpy as jnp
from jax import lax
from jax.experimental import pallas as pl
from jax.experimental.pallas import tpu as pltpu
from jax.experimental.pallas import tpu_sc as plsc

N_NODES = 10000
N_EDGES = 320000
D = 128

NC = 2
NS = 16
CHUNK = 80
EDGES_PER_TILE = N_EDGES // (NC * NS)
N_CHUNKS = EDGES_PER_TILE // CHUNK
TABLE_ROWS = 10240
ROWS_PER_TILE = TABLE_ROWS // NS
NB = 3


def _seg_sum_kernel(temp_hbm, edges_hbm, zeros_hbm, out_hbm,
                    rows0, rows1, rows2, sb0, sb1, sb2, db0, db1, db2,
                    table_sh, g0, g1, g2, s0, s1, s2, x0, x1, x2):
    c = lax.axis_index("c")
    s = lax.axis_index("s")
    wid = c * NS + s

    rows = (rows0, rows1, rows2)
    sidx = (sb0, sb1, sb2)
    dstb = (db0, db1, db2)
    gsem = (g0, g1, g2)
    ssem = (s0, s1, s2)
    xsem = (x0, x1, x2)

    def _off(i, base):
        return pl.multiple_of(base + wid * EDGES_PER_TILE + i * CHUNK, 8)

    def load_src(i, b):
        pltpu.async_copy(edges_hbm.at[pl.ds(_off(i, 0), CHUNK)],
                         sidx[b], xsem[b])

    def wait_src(b):
        pltpu.make_async_copy(edges_hbm.at[pl.ds(0, CHUNK)],
                              sidx[b], xsem[b]).wait()

    def load_dst(i, b):
        pltpu.async_copy(edges_hbm.at[pl.ds(_off(i, N_EDGES), CHUNK)],
                         dstb[b], gsem[b])

    def issue_gather(b):
        pltpu.async_copy(temp_hbm.at[sidx[b]], rows[b], gsem[b])

    def wait_gather(b):
        pltpu.make_async_copy(edges_hbm.at[pl.ds(0, CHUNK)],
                              dstb[b], gsem[b]).wait()
        pltpu.make_async_copy(temp_hbm.at[pl.ds(0, CHUNK)],
                              rows[b], gsem[b]).wait()

    def start_scatter(b):
        pltpu.async_copy(rows[b], table_sh.at[dstb[b]], ssem[b], add=True)

    def wait_scatter(b):
        pltpu.make_async_copy(rows[b], table_sh.at[dstb[b]], ssem[b]).wait()

    for b in range(NB):
        load_src(b, b)
    for b in range(2):
        load_dst(b, b)
        wait_src(b)
        issue_gather(b)

    pltpu.sync_copy(zeros_hbm, rows2)
    for k in range(ROWS_PER_TILE // CHUNK):
        pltpu.sync_copy(rows2,
                        table_sh.at[pl.ds(s * ROWS_PER_TILE + k * CHUNK,
                                          CHUNK)])
    plsc.subcore_barrier()

    def step(i, b, first, last):
        wait_gather(b)
        start_scatter(b)
        if not first:
            wait_scatter((b + 2) % NB)
        if last:
            return
        bn = (b + 2) % NB

        def load_src_ahead():
            load_src(i + NB, b)

        def issue_ahead():
            load_dst(i + 2, bn)
            wait_src(bn)
            issue_gather(bn)

        if isinstance(i, int):
            if i + NB < N_CHUNKS:
                load_src_ahead()
            if i + 2 < N_CHUNKS:
                issue_ahead()
        else:
            pl.when(i + NB < N_CHUNKS)(load_src_ahead)
            pl.when(i + 2 < N_CHUNKS)(issue_ahead)

    step(0, 0, True, False)

    def triple(k, carry):
        i = 3 * k
        step(i + 1, 1, False, False)
        step(i + 2, 2, False, False)
        step(i + 3, 0, False, False)
        return carry

    lax.fori_loop(0, (N_CHUNKS - 2) // 3, triple, 0)
    step(N_CHUNKS - 1, (N_CHUNKS - 1) % NB, False, True)
    wait_scatter((N_CHUNKS - 1) % NB)

    plsc.subcore_barrier()

    NW = ROWS_PER_TILE // CHUNK

    def _row0(k):
        return s * ROWS_PER_TILE + k * CHUNK

    def read_tbl(k, b):
        pltpu.async_copy(table_sh.at[pl.ds(_row0(k), CHUNK)],
                         rows[b], gsem[b])

    def wait_read(b):
        pltpu.make_async_copy(table_sh.at[pl.ds(0, CHUNK)],
                              rows[b], gsem[b]).wait()

    def write_out(k, b):
        pltpu.async_copy(rows[b], out_hbm.at[c, pl.ds(_row0(k), CHUNK)],
                         ssem[b])

    def wait_write(b):
        pltpu.make_async_copy(rows[b], out_hbm.at[c, pl.ds(0, CHUNK)],
                              ssem[b]).wait()

    read_tbl(0, 0)
    for k in range(NW):
        b = k % 2
        if k + 1 < NW:
            if k >= 1:
                wait_write(1 - b)
            read_tbl(k + 1, 1 - b)
        wait_read(b)
        write_out(k, b)
    wait_write(0)
    wait_write(1)


def _seg_sum(temp, edges_flat, zeros_stage):
    mesh = plsc.VectorSubcoreMesh(core_axis_name="c", subcore_axis_name="s",
                                  num_cores=NC, num_subcores=NS)
    kern = pl.kernel(
        _seg_sum_kernel,
        out_type=jax.ShapeDtypeStruct((NC, TABLE_ROWS, D), jnp.float32),
        mesh=mesh,
        scratch_types=(
            [pltpu.VMEM((CHUNK, D), jnp.float32) for _ in range(NB)]
            + [pltpu.VMEM((CHUNK,), jnp.int32) for _ in range(2 * NB)]
            + [pltpu.VMEM_SHARED((TABLE_ROWS, D), jnp.float32)]
            + [pltpu.SemaphoreType.DMA for _ in range(3 * NB)]
        ),
    )
    return kern(temp, edges_flat, zeros_stage)


ROW_BLK = 2000
D_X = 96
D_NZ = 16


def _prep_kernel(xt_ref, et_ref, ht_ref, o_ref):
    x = jnp.swapaxes(xt_ref[...], 0, 1)
    e = jnp.swapaxes(et_ref[...], 0, 1)
    h = jnp.swapaxes(ht_ref[...], 0, 1)
    o_ref[...] = jnp.concatenate([x, e, h], axis=1)


def _prep_temp(X, epsilon, h):
    return pl.pallas_call(
        _prep_kernel,
        out_shape=jax.ShapeDtypeStruct((N_NODES, D), jnp.float32),
    )(X.T, epsilon.T, h.T)


def _lin_body(t_ref, a0_ref, a1_ref, w_ref, b_ref, sc_ref):
    x = sc_ref[...] * t_ref[...] + a0_ref[0] + a1_ref[0]
    y = jnp.dot(x, w_ref[...], preferred_element_type=jnp.float32)
    return jnp.maximum(y + b_ref[...], 0.0)


def _lin_kernel(t_ref, a0_ref, a1_ref, w_ref, b_ref, sc_ref, o_ref):
    o_ref[...] = _lin_body(t_ref, a0_ref, a1_ref, w_ref, b_ref, sc_ref)


_LIN_IN_SPECS = [
    pl.BlockSpec((ROW_BLK, D), lambda i: (i, 0)),
    pl.BlockSpec((1, ROW_BLK, D), lambda i: (0, i, 0)),
    pl.BlockSpec((1, ROW_BLK, D), lambda i: (1, i, 0)),
    pl.BlockSpec((D, D), lambda i: (0, 0)),
    pl.BlockSpec((1, D), lambda i: (0, 0)),
    pl.BlockSpec((1, D), lambda i: (0, 0)),
]


def _lin_layer(temp, agg2, w, b_row, scale_row):
    return pl.pallas_call(
        _lin_kernel,
        grid=(N_NODES // ROW_BLK,),
        in_specs=_LIN_IN_SPECS,
        out_specs=pl.BlockSpec((ROW_BLK, D), lambda i: (i, 0)),
        out_shape=jax.ShapeDtypeStruct((N_NODES, D), jnp.float32),
    )(temp, agg2, agg2, w, b_row, scale_row)


def _lin2_kernel(t_ref, a0_ref, a1_ref, w_ref, b_ref, sc_ref, o_ref, acc_ref):
    i = pl.program_id(0)

    @pl.when(i == 0)
    def _():
        acc_ref[...] = jnp.zeros_like(acc_ref)

    y = _lin_body(t_ref, a0_ref, a1_ref, w_ref, b_ref, sc_ref)
    acc_ref[...] += jnp.sum(y, axis=0, keepdims=True)

    @pl.when(i == pl.num_programs(0) - 1)
    def _():
        o_ref[...] = jnp.maximum(acc_ref[...] * (1.0 / N_NODES), 0.0)


def _lin_layer2_readout(temp, agg2, w, b_row, scale_row):
    return pl.pallas_call(
        _lin2_kernel,
        grid=(N_NODES // ROW_BLK,),
        in_specs=_LIN_IN_SPECS,
        out_specs=pl.BlockSpec((1, D), lambda i: (0, 0)),
        out_shape=jax.ShapeDtypeStruct((1, D), jnp.float32),
        scratch_shapes=[pltpu.VMEM((1, D), jnp.float32)],
    )(temp, agg2, agg2, w, b_row, scale_row)


def kernel(X, h, epsilon, edge_index, W1, b1, W2, b2, eps0, eps1):
    temp = _prep_temp(X, epsilon, h)
    edges_flat = edge_index.reshape(2 * N_EDGES)
    zeros_stage = jnp.zeros((CHUNK, D), jnp.float32)
    b1r = b1.reshape(1, D)
    b2r = b2.reshape(1, D)
    sc0 = jnp.full((1, D), 1.0, jnp.float32) + eps0
    sc1 = jnp.full((1, D), 1.0, jnp.float32) + eps1

    agg2 = _seg_sum(temp, edges_flat, zeros_stage)
    y1 = _lin_layer(temp, agg2, W1, b1r, sc0)
    agg2 = _seg_sum(y1, edges_flat, zeros_stage)
    out = _lin_layer2_readout(y1, agg2, W2, b2r, sc1)
    return (out.reshape(D), epsilon)
